# baseline (speedup 1.0000x reference)
import functools

import jax
import jax.numpy as jnp
from jax import lax
from jax.experimental import pallas as pl
from jax.experimental.pallas import tpu as pltpu
from jax.experimental.pallas import tpu_sc as plsc

_NBUF = 8
_SPAD = 56


def _make_sc_gather(V, D, B):
    info = plsc.get_sparse_core_info()
    nw = info.num_cores * info.num_subcores
    spw = B // nw
    n_outer = spw // _NBUF
    group = _NBUF * _SPAD
    assert spw % _NBUF == 0
    mesh = plsc.VectorSubcoreMesh(core_axis_name="c", subcore_axis_name="s")

    @functools.partial(
        pl.kernel,
        mesh=mesh,
        out_type=jax.ShapeDtypeStruct((B * _SPAD, D), jnp.float32),
        scratch_types=[
            pltpu.VMEM((spw, 128), jnp.int32),
            pltpu.VMEM((group, D), jnp.float32),
        ] + [pltpu.SemaphoreType.DMA] * _NBUF,
        compiler_params=pltpu.CompilerParams(use_tc_tiling_on_sc=False),
    )
    def gather(idx_hbm, table_hbm, out_hbm, idx_v, rows_v, *sems):
        wid = lax.axis_index("s") * info.num_cores + lax.axis_index("c")
        sample_base = wid * spw
        pltpu.sync_copy(idx_hbm.at[pl.ds(sample_base, spw)], idx_v)

        def body(g, carry):
            s0 = g * _NBUF
            cps = []
            for j in range(_NBUF):
                cps.append(pltpu.async_copy(
                    table_hbm.at[idx_v.at[s0 + j, pl.ds(0, _SPAD)]],
                    rows_v.at[pl.ds(j * _SPAD, _SPAD)],
                    sems[j]))
            for cp in cps:
                cp.wait()
            pltpu.sync_copy(
                rows_v,
                out_hbm.at[pl.ds((sample_base + s0) * _SPAD, group)])
            return carry

        lax.fori_loop(0, n_outer, body, 0)

    return gather


def _make_tc_head(Bb, K, C, BB):
    def body(e_ref, w_ref, b_ref, o_ref):
        e = e_ref[...].astype(jnp.bfloat16)
        logits = lax.dot_general(e, w_ref[...], (((1,), (1,)), ((), ())),
                                 preferred_element_type=jnp.float32)
        logits = logits + b_ref[...]
        m = jnp.max(logits, axis=-1, keepdims=True)
        p = jnp.exp(logits - m)
        o_ref[...] = p / jnp.sum(p, axis=-1, keepdims=True)

    return pl.pallas_call(
        body,
        grid=(Bb // BB,),
        in_specs=[
            pl.BlockSpec((BB, K), lambda i: (i, 0)),
            pl.BlockSpec((C, K), lambda i: (0, 0)),
            pl.BlockSpec((1, C), lambda i: (0, 0)),
        ],
        out_specs=pl.BlockSpec((BB, C), lambda i: (i, 0)),
        out_shape=jax.ShapeDtypeStruct((Bb, C), jnp.float32),
    )


def kernel(x, embed, W, b):
    B, S = x.shape
    V, D = embed.shape
    C = W.shape[0]
    k_pad = _SPAD * D

    x128 = jnp.pad(x.astype(jnp.int32), ((0, 0), (0, 128 - S)))

    w3 = W.reshape(C, S, D)
    w_pad = jnp.concatenate(
        [w3, jnp.zeros((C, _SPAD - S, D), W.dtype)], axis=1)
    w_pad = w_pad.reshape(C, k_pad).astype(jnp.bfloat16)

    e = _make_sc_gather(V, D, B)(x128, embed)
    e2 = e.reshape(B, k_pad)
    head = _make_tc_head(B, k_pad, C, 512)
    return head(e2, w_pad, b.reshape(1, C))
